# baseline (speedup 1.0000x reference)
import functools

import jax
import jax.numpy as jnp
from jax import lax
from jax.experimental import pallas as pl
from jax.experimental.pallas import tpu as pltpu
from jax.experimental.pallas import tpu_sc as plsc

BATCH = 16384
EMBED = 64
LANES = 16
NUM_CORES = 2
NUM_SUBCORES = 16
NUM_WORKERS = NUM_CORES * NUM_SUBCORES
B_PER_W = BATCH // NUM_WORKERS
GATHER_CHUNK = 128
N_CHUNKS = B_PER_W // GATHER_CHUNK
GROUPS = B_PER_W // LANES


def _sc_body(uids_hbm, iids_hbm, utab_hbm, itab_hbm, out_hbm,
             uidx_v, iidx_v, urows_v, irows_v, out_v, sem_u, sem_i):
    wid = lax.axis_index("s") * NUM_CORES + lax.axis_index("c")
    base = wid * B_PER_W

    pltpu.sync_copy(uids_hbm.at[pl.ds(base, B_PER_W)], uidx_v)
    pltpu.sync_copy(iids_hbm.at[pl.ds(base, B_PER_W)], iidx_v)

    copies = []
    for c in range(N_CHUNKS):
        sl = pl.ds(c * GATHER_CHUNK, GATHER_CHUNK)
        copies.append(pltpu.async_copy(
            utab_hbm.at[uidx_v.at[sl]], urows_v.at[sl], sem_u))
        copies.append(pltpu.async_copy(
            itab_hbm.at[iidx_v.at[sl]], irows_v.at[sl], sem_i))
    for cp in copies:
        cp.wait()

    lane = lax.iota(jnp.int32, LANES)

    def group_body(g, carry):
        r0 = g * LANES
        tot = jnp.zeros((LANES,), jnp.float32)
        for j in range(LANES):
            r = r0 + j
            acc = (urows_v[r, pl.ds(0, 16)] * irows_v[r, pl.ds(0, 16)]
                   + urows_v[r, pl.ds(16, 16)] * irows_v[r, pl.ds(16, 16)])
            acc = acc + (urows_v[r, pl.ds(32, 16)] * irows_v[r, pl.ds(32, 16)]
                         + urows_v[r, pl.ds(48, 16)] * irows_v[r, pl.ds(48, 16)])
            tot = jnp.where(lane == j, jnp.sum(acc), tot)
        out_v[pl.ds(r0, LANES)] = tot
        return carry

    lax.fori_loop(0, GROUPS, group_body, 0)

    pltpu.sync_copy(out_v, out_hbm.at[pl.ds(base, B_PER_W)])


def kernel(user_ids, item_ids, user_table, item_table):
    mesh = plsc.VectorSubcoreMesh(core_axis_name="c", subcore_axis_name="s")
    run = functools.partial(
        pl.kernel,
        mesh=mesh,
        compiler_params=pltpu.CompilerParams(
            needs_layout_passes=False, use_tc_tiling_on_sc=False),
        out_type=jax.ShapeDtypeStruct((BATCH,), jnp.float32),
        scratch_types=[
            pltpu.VMEM((B_PER_W,), jnp.int32),
            pltpu.VMEM((B_PER_W,), jnp.int32),
            pltpu.VMEM((B_PER_W, EMBED), jnp.float32),
            pltpu.VMEM((B_PER_W, EMBED), jnp.float32),
            pltpu.VMEM((B_PER_W,), jnp.float32),
            pltpu.SemaphoreType.DMA,
            pltpu.SemaphoreType.DMA,
        ],
    )(_sc_body)
    return run(user_ids.astype(jnp.int32), item_ids.astype(jnp.int32),
               user_table, item_table)
